# Initial kernel scaffold; baseline (speedup 1.0000x reference)
import jax
import jax.numpy as jnp
from jax.experimental import pallas as pl


def kernel(h, edge_index, r, norm, emb, W1, loop_w1, b1, W2, loop_w2, b2, W3, loop_w3, b3):
    raise NotImplementedError("write your pallas kernel here")


# speedup vs baseline: 9.5007x; 9.5007x over previous
import functools

import jax
import jax.numpy as jnp
from jax import lax
from jax.experimental import pallas as pl
from jax.experimental.pallas import tpu as pltpu
from jax.experimental.pallas import tpu_sc as plsc

N = 10000
E = 160000
D = 256
R = 64
NB = 4
SUB = D // NB

NC = 2
NS = 16
NW = NC * NS

HALF = N // 2
HALFP = 5120
NP = 2 * HALFP
PADGAP = HALFP - HALF
WROWS = 320

TE = 512
EP = 163840
RPW_E = EP // NW
RPW_N = NP // NW
EPT = EP // NS


def _sc_gather(table, idx, rpw, ch):
    b = idx.shape[0]
    nch = rpw // ch
    mesh = plsc.VectorSubcoreMesh(core_axis_name="c", subcore_axis_name="s")

    @functools.partial(
        pl.kernel,
        out_type=jax.ShapeDtypeStruct((b, D), jnp.float32),
        mesh=mesh,
        scratch_types=[
            pltpu.VMEM((ch,), jnp.int32),
            pltpu.VMEM((ch, D), jnp.float32),
            pltpu.SemaphoreType.DMA,
        ],
    )
    def k(table_hbm, idx_hbm, out_hbm, idx_v, rows_v, sem):
        wid = lax.axis_index("s") * NC + lax.axis_index("c")

        def body(g, carry):
            base = wid * rpw + g * ch
            pltpu.sync_copy(idx_hbm.at[pl.ds(base, ch)], idx_v)
            pltpu.async_copy(table_hbm.at[idx_v], rows_v, sem).wait()
            pltpu.sync_copy(rows_v, out_hbm.at[pl.ds(base, ch)])
            return carry

        lax.fori_loop(0, nch, body, 0)

    return k(table, idx)


OUTR = NP + TE


def _tc_aggregate(msgd, dstm_row, base):
    nt = EP // TE

    def body(m_ref, d_ref, base_ref, o_ref):
        t = pl.program_id(0)

        @pl.when(t == 0)
        def _():
            o_ref[pl.ds(0, NP), :] = base_ref[...]
            o_ref[pl.ds(NP, TE), :] = jnp.zeros((TE, D), jnp.float32)

        m = m_ref[...]
        dv = d_ref[...]
        wb0 = (jnp.min(dv) // 8) * 8
        nw = (jnp.max(dv) - wb0) // TE + 1
        ii = lax.broadcasted_iota(jnp.int32, (TE, 1), 0)

        def wbody(w, carry):
            wb = wb0 + w * TE
            pt = jnp.where((ii + wb) == dv, 1.0, 0.0)
            contrib = jnp.dot(pt, m, preferred_element_type=jnp.float32)
            o_ref[pl.ds(wb, TE), :] += contrib
            return carry

        lax.fori_loop(0, nw, wbody, 0)

    return pl.pallas_call(
        body,
        grid=(nt,),
        in_specs=[
            pl.BlockSpec((TE, D), lambda t: (t, 0)),
            pl.BlockSpec((1, TE), lambda t: (0, t)),
            pl.BlockSpec((NP, D), lambda t: (0, 0)),
        ],
        out_specs=pl.BlockSpec((OUTR, D), lambda t: (0, 0)),
        out_shape=jax.ShapeDtypeStruct((OUTR, D), jnp.float32),
    )(msgd, dstm_row, base)


def _tc_blockmm(xs, r_col, norm_col, wbd, relu_in):
    nt = EP // TE

    def body(xs_ref, r_ref, nm_ref, w_ref, o_ref):
        x = xs_ref[...]
        if relu_in:
            x = jnp.maximum(x, 0.0)
        x = x * nm_ref[...]
        rv = r_ref[...]
        lo = jnp.min(rv)
        hi = jnp.max(rv)

        def rel_body(rr, acc):
            xm = jnp.where(rv == rr, x, 0.0)
            return acc + jnp.dot(xm, w_ref[rr],
                                 preferred_element_type=jnp.float32)

        o_ref[...] = lax.fori_loop(lo, hi + 1, rel_body,
                                   jnp.zeros((TE, D), jnp.float32))

    return pl.pallas_call(
        body,
        grid=(nt,),
        in_specs=[
            pl.BlockSpec((TE, D), lambda t: (t, 0)),
            pl.BlockSpec((TE, 1), lambda t: (t, 0)),
            pl.BlockSpec((TE, 1), lambda t: (t, 0)),
            pl.BlockSpec((R, D, D), lambda t: (0, 0, 0)),
        ],
        out_specs=pl.BlockSpec((TE, D), lambda t: (t, 0)),
        out_shape=jax.ShapeDtypeStruct((EP, D), jnp.float32),
    )(xs, r_col, norm_col, wbd)


def _tc_dense(x, wbs, relu_in):
    nt = NP // TE
    no = len(wbs)

    def body(*refs):
        x_ref = refs[0]
        x = x_ref[...]
        if relu_in:
            x = jnp.maximum(x, 0.0)
        for i in range(no):
            w_ref = refs[1 + 2 * i]
            b_ref = refs[2 + 2 * i]
            o_ref = refs[1 + 2 * no + i]
            o_ref[...] = jnp.dot(x, w_ref[...],
                                 preferred_element_type=jnp.float32) + b_ref[...]

    in_specs = [pl.BlockSpec((TE, D), lambda t: (t, 0))]
    args = [x]
    for (w, b) in wbs:
        in_specs.append(pl.BlockSpec((D, D), lambda t: (0, 0)))
        in_specs.append(pl.BlockSpec((1, D), lambda t: (0, 0)))
        args.append(w)
        args.append(b.reshape(1, D))
    out = pl.pallas_call(
        body,
        grid=(nt,),
        in_specs=in_specs,
        out_specs=[pl.BlockSpec((TE, D), lambda t: (t, 0))] * no,
        out_shape=[jax.ShapeDtypeStruct((NP, D), jnp.float32)] * no,
    )(*args)
    return out


def _tc_combine(z_mean, z_log_std, eps):
    nt = (N + TE - 1) // TE

    def body(a_ref, b_ref, e_ref, o_ref):
        o_ref[...] = a_ref[...] + b_ref[...] * e_ref[...]

    return pl.pallas_call(
        body,
        grid=(nt,),
        in_specs=[pl.BlockSpec((TE, D), lambda t: (t, 0))] * 3,
        out_specs=pl.BlockSpec((TE, D), lambda t: (t, 0)),
        out_shape=jax.ShapeDtypeStruct((N, D), jnp.float32),
    )(z_mean, z_log_std, eps)


def _bdiag(w):
    eye = jnp.eye(NB, dtype=jnp.float32)
    return jnp.einsum('rnij,nm->rnimj', w, eye).reshape(R, D, D)


def kernel(h, edge_index, r, norm, emb,
           W1, loop_w1, b1, W2, loop_w2, b2, W3, loop_w3, b3):
    src = edge_index[0].astype(jnp.int32)
    dst = edge_index[1].astype(jnp.int32)
    r = r.astype(jnp.int32)

    perm = jnp.argsort(r)
    pad = EP - E
    src_p = jnp.concatenate([src[perm], jnp.zeros((pad,), jnp.int32)])
    r_p = jnp.concatenate([r[perm], jnp.full((pad,), R - 1, jnp.int32)])
    norm_p = jnp.concatenate([norm[perm, 0], jnp.zeros((pad,), jnp.float32)])
    r_col = r_p.reshape(EP, 1)
    norm_col = norm_p.reshape(EP, 1)

    perm_d = jnp.argsort(dst)
    inv_r = jnp.zeros((E,), jnp.int32).at[perm].set(
        jnp.arange(E, dtype=jnp.int32))
    gidx = jnp.concatenate([inv_r[perm_d], jnp.full((pad,), E, jnp.int32)])
    dst_d = dst[perm_d]
    dstm_row = jnp.concatenate(
        [jnp.where(dst_d >= HALF, dst_d + PADGAP, dst_d),
         jnp.full((pad,), NP, jnp.int32)]).reshape(1, EP)

    src_m = jnp.where(src_p >= HALF, src_p + PADGAP, src_p)
    rows = jnp.arange(NP, dtype=jnp.int32)
    n_of_row = jnp.where(rows >= HALFP, rows - PADGAP, rows)
    n_of_row = jnp.minimum(n_of_row, N - 1)
    h_m = h.astype(jnp.int32)[n_of_row]

    wbd1 = _bdiag(W1)
    wbd2 = _bdiag(W2)
    wbd3 = _bdiag(W3)

    x0 = _sc_gather(emb, h_m, RPW_N, 64)
    (base1,) = _tc_dense(x0, [(loop_w1, b1)], relu_in=False)
    xs0 = _sc_gather(x0, src_m, RPW_E, 128)
    msg1 = _tc_blockmm(xs0, r_col, norm_col, wbd1, relu_in=False)
    msgd1 = _sc_gather(msg1, gidx, RPW_E, 128)
    out1 = _tc_aggregate(msgd1, dstm_row, base1)[:NP]

    base2, base3 = _tc_dense(out1, [(loop_w2, b2), (loop_w3, b3)],
                             relu_in=True)
    xs1 = _sc_gather(out1, src_m, RPW_E, 128)
    msg2 = _tc_blockmm(xs1, r_col, norm_col, wbd2, relu_in=True)
    msg3 = _tc_blockmm(xs1, r_col, norm_col, wbd3, relu_in=True)
    msgd2 = _sc_gather(msg2, gidx, RPW_E, 128)
    msgd3 = _sc_gather(msg3, gidx, RPW_E, 128)
    out2 = _tc_aggregate(msgd2, dstm_row, base2)[:NP]
    out3 = _tc_aggregate(msgd3, dstm_row, base3)[:NP]

    eps = jax.random.normal(jax.random.key(42), (N, D), dtype=jnp.float32)
    z2 = jnp.concatenate([out2[:HALF], out2[HALFP:HALFP + HALF]], axis=0)
    z3 = jnp.concatenate([out3[:HALF], out3[HALFP:HALFP + HALF]], axis=0)
    return _tc_combine(z2, z3, eps)


# speedup vs baseline: 10.8867x; 1.1459x over previous
import functools

import jax
import jax.numpy as jnp
from jax import lax
from jax.experimental import pallas as pl
from jax.experimental.pallas import tpu as pltpu
from jax.experimental.pallas import tpu_sc as plsc

N = 10000
E = 160000
D = 256
R = 64
NB = 4
SUB = D // NB

NC = 2
NS = 16
NW = NC * NS

HALF = N // 2
HALFP = 5120
NP = 2 * HALFP
PADGAP = HALFP - HALF
WROWS = 320

TE = 512
EP = 163840
RPW_E = EP // NW
RPW_N = NP // NW
EPT = EP // NS


def _sc_gather(table, idx, rpw, ch):
    b = idx.shape[0]
    w = table.shape[1]
    nch = rpw // ch
    mesh = plsc.VectorSubcoreMesh(core_axis_name="c", subcore_axis_name="s")
    nouter = (nch + 2 + 2) // 3

    @functools.partial(
        pl.kernel,
        out_type=jax.ShapeDtypeStruct((b, w), jnp.float32),
        mesh=mesh,
        scratch_types=[
            pltpu.VMEM((3, ch), jnp.int32),
            pltpu.VMEM((3, ch, w), jnp.float32),
            pltpu.SemaphoreType.DMA,
            pltpu.SemaphoreType.DMA,
            pltpu.SemaphoreType.DMA,
            pltpu.SemaphoreType.DMA,
            pltpu.SemaphoreType.DMA,
            pltpu.SemaphoreType.DMA,
        ],
    )
    def k(table_hbm, idx_hbm, out_hbm, idx_v, rows_v,
          gs0, gs1, gs2, ws0, ws1, ws2):
        gsems = [gs0, gs1, gs2]
        wsems = [ws0, ws1, ws2]
        wid = lax.axis_index("s") * NC + lax.axis_index("c")
        base0 = wid * rpw

        def outer(go, carry):
            for bs in range(3):
                gg = go * 3 + bs

                @pl.when(gg < nch)
                def _():
                    @pl.when(gg >= 3)
                    def _():
                        pltpu.make_async_copy(
                            rows_v.at[bs], out_hbm.at[pl.ds(0, ch)],
                            wsems[bs]).wait()

                    pltpu.sync_copy(
                        idx_hbm.at[pl.ds(base0 + gg * ch, ch)],
                        idx_v.at[bs])
                    pltpu.async_copy(
                        table_hbm.at[idx_v.at[bs]], rows_v.at[bs],
                        gsems[bs])

                g2 = gg - 2
                b2 = (bs + 1) % 3

                @pl.when((g2 >= 0) & (g2 < nch))
                def _():
                    pltpu.make_async_copy(
                        table_hbm.at[idx_v.at[b2]], rows_v.at[b2],
                        gsems[b2]).wait()
                    pltpu.async_copy(
                        rows_v.at[b2],
                        out_hbm.at[pl.ds(base0 + g2 * ch, ch)],
                        wsems[b2])
            return carry

        lax.fori_loop(0, nouter, outer, 0)
        for bs in range(3):
            pltpu.make_async_copy(
                rows_v.at[bs], out_hbm.at[pl.ds(0, ch)], wsems[bs]).wait()

    return k(table, idx)


OUTR = NP + TE


def _tc_aggregate(msgd, dstm_row, base):
    nt = EP // TE
    dw = msgd.shape[1]

    def body(m_ref, d_ref, *rest):
        if base is None:
            (o_ref,) = rest
        else:
            base_ref, o_ref = rest
        t = pl.program_id(0)

        @pl.when(t == 0)
        def _():
            if base is None:
                o_ref[...] = jnp.zeros((OUTR, dw), jnp.float32)
            else:
                o_ref[pl.ds(0, NP), :] = base_ref[...]
                o_ref[pl.ds(NP, TE), :] = jnp.zeros((TE, dw), jnp.float32)

        m = m_ref[...]
        dv = d_ref[...]
        wb0 = (jnp.min(dv) // 8) * 8
        nw = (jnp.max(dv) - wb0) // TE + 1
        ii = lax.broadcasted_iota(jnp.int32, (TE, 1), 0)

        def wbody(w, carry):
            wb = wb0 + w * TE
            pt = jnp.where((ii + wb) == dv, 1.0, 0.0)
            contrib = jnp.dot(pt, m, preferred_element_type=jnp.float32)
            o_ref[pl.ds(wb, TE), :] += contrib
            return carry

        lax.fori_loop(0, nw, wbody, 0)

    in_specs = [
        pl.BlockSpec((TE, dw), lambda t: (t, 0)),
        pl.BlockSpec((1, TE), lambda t: (0, t)),
    ]
    args = [msgd, dstm_row]
    if base is not None:
        in_specs.append(pl.BlockSpec((NP, dw), lambda t: (0, 0)))
        args.append(base)
    return pl.pallas_call(
        body,
        grid=(nt,),
        in_specs=in_specs,
        out_specs=pl.BlockSpec((OUTR, dw), lambda t: (0, 0)),
        out_shape=jax.ShapeDtypeStruct((OUTR, dw), jnp.float32),
    )(*args)


def _tc_blockmm(xs, r_col, norm_col, wbd, relu_in):
    nt = EP // TE
    dw = wbd.shape[2]

    def body(xs_ref, r_ref, nm_ref, w_ref, o_ref):
        x = xs_ref[...]
        if relu_in:
            x = jnp.maximum(x, 0.0)
        x = x * nm_ref[...]
        rv = r_ref[...]
        lo = jnp.min(rv)
        hi = jnp.max(rv)

        def rel_body(rr, acc):
            xm = jnp.where(rv == rr, x, 0.0)
            return acc + jnp.dot(xm, w_ref[rr],
                                 preferred_element_type=jnp.float32)

        o_ref[...] = lax.fori_loop(lo, hi + 1, rel_body,
                                   jnp.zeros((TE, dw), jnp.float32))

    return pl.pallas_call(
        body,
        grid=(nt,),
        in_specs=[
            pl.BlockSpec((TE, D), lambda t: (t, 0)),
            pl.BlockSpec((TE, 1), lambda t: (t, 0)),
            pl.BlockSpec((TE, 1), lambda t: (t, 0)),
            pl.BlockSpec((R, D, dw), lambda t: (0, 0, 0)),
        ],
        out_specs=pl.BlockSpec((TE, dw), lambda t: (t, 0)),
        out_shape=jax.ShapeDtypeStruct((EP, dw), jnp.float32),
    )(xs, r_col, norm_col, wbd)


def _tc_dense(x, wbs, relu_in):
    nt = NP // TE
    no = len(wbs)

    def body(*refs):
        x_ref = refs[0]
        x = x_ref[...]
        if relu_in:
            x = jnp.maximum(x, 0.0)
        for i in range(no):
            w_ref = refs[1 + 2 * i]
            b_ref = refs[2 + 2 * i]
            o_ref = refs[1 + 2 * no + i]
            o_ref[...] = jnp.dot(x, w_ref[...],
                                 preferred_element_type=jnp.float32) + b_ref[...]

    in_specs = [pl.BlockSpec((TE, D), lambda t: (t, 0))]
    args = [x]
    for (w, b) in wbs:
        in_specs.append(pl.BlockSpec((D, D), lambda t: (0, 0)))
        in_specs.append(pl.BlockSpec((1, D), lambda t: (0, 0)))
        args.append(w)
        args.append(b.reshape(1, D))
    out = pl.pallas_call(
        body,
        grid=(nt,),
        in_specs=in_specs,
        out_specs=[pl.BlockSpec((TE, D), lambda t: (t, 0))] * no,
        out_shape=[jax.ShapeDtypeStruct((NP, D), jnp.float32)] * no,
    )(*args)
    return out


def _tc_combine(agg_mean, base_mean, agg_lstd, base_lstd, eps):
    nt = (N + TE - 1) // TE

    def body(a_ref, ab_ref, b_ref, bb_ref, e_ref, o_ref):
        o_ref[...] = (a_ref[...] + ab_ref[...]
                      + (b_ref[...] + bb_ref[...]) * e_ref[...])

    return pl.pallas_call(
        body,
        grid=(nt,),
        in_specs=[pl.BlockSpec((TE, D), lambda t: (t, 0))] * 5,
        out_specs=pl.BlockSpec((TE, D), lambda t: (t, 0)),
        out_shape=jax.ShapeDtypeStruct((N, D), jnp.float32),
    )(agg_mean, base_mean, agg_lstd, base_lstd, eps)


def _bdiag(w):
    eye = jnp.eye(NB, dtype=jnp.float32)
    return jnp.einsum('rnij,nm->rnimj', w, eye).reshape(R, D, D)


def kernel(h, edge_index, r, norm, emb,
           W1, loop_w1, b1, W2, loop_w2, b2, W3, loop_w3, b3):
    src = edge_index[0].astype(jnp.int32)
    dst = edge_index[1].astype(jnp.int32)
    r = r.astype(jnp.int32)

    perm = jnp.argsort(r)
    pad = EP - E
    src_p = jnp.concatenate([src[perm], jnp.zeros((pad,), jnp.int32)])
    r_p = jnp.concatenate([r[perm], jnp.full((pad,), R - 1, jnp.int32)])
    norm_p = jnp.concatenate([norm[perm, 0], jnp.zeros((pad,), jnp.float32)])
    r_col = r_p.reshape(EP, 1)
    norm_col = norm_p.reshape(EP, 1)

    perm_d = jnp.argsort(dst)
    inv_r = jnp.zeros((E,), jnp.int32).at[perm].set(
        jnp.arange(E, dtype=jnp.int32))
    gidx = jnp.concatenate([inv_r[perm_d], jnp.full((pad,), E, jnp.int32)])
    dst_d = dst[perm_d]
    dstm_row = jnp.concatenate(
        [jnp.where(dst_d >= HALF, dst_d + PADGAP, dst_d),
         jnp.full((pad,), NP, jnp.int32)]).reshape(1, EP)

    src_m = jnp.where(src_p >= HALF, src_p + PADGAP, src_p)
    rows = jnp.arange(NP, dtype=jnp.int32)
    n_of_row = jnp.where(rows >= HALFP, rows - PADGAP, rows)
    n_of_row = jnp.minimum(n_of_row, N - 1)
    h_m = h.astype(jnp.int32)[n_of_row]

    wbd1 = _bdiag(W1)
    wbd2 = _bdiag(W2)
    wbd3 = _bdiag(W3)

    x0 = _sc_gather(emb, h_m, RPW_N, 64)
    (base1,) = _tc_dense(x0, [(loop_w1, b1)], relu_in=False)
    xs0 = _sc_gather(x0, src_m, RPW_E, 128)
    msg1 = _tc_blockmm(xs0, r_col, norm_col, wbd1, relu_in=False)
    msgd1 = _sc_gather(msg1, gidx, RPW_E, 128)
    out1 = _tc_aggregate(msgd1, dstm_row, base1)[:NP]

    base2, base3 = _tc_dense(out1, [(loop_w2, b2), (loop_w3, b3)],
                             relu_in=True)
    xs1 = _sc_gather(out1, src_m, RPW_E, 128)
    msg23 = _tc_blockmm(xs1, r_col, norm_col,
                        jnp.concatenate([wbd2, wbd3], axis=2), relu_in=True)
    msgd23 = _sc_gather(msg23, gidx, RPW_E, 64)
    out23 = _tc_aggregate(msgd23, dstm_row, None)[:NP]

    def unm(a):
        return jnp.concatenate([a[:HALF], a[HALFP:HALFP + HALF]], axis=0)

    eps = jax.random.normal(jax.random.key(42), (N, D), dtype=jnp.float32)
    return _tc_combine(unm(out23[:, :D]), unm(base2),
                       unm(out23[:, D:]), unm(base3), eps)
